# baseline (speedup 1.0000x reference)
import jax
import jax.numpy as jnp
from jax.experimental import pallas as pl
from jax.experimental.pallas import tpu as pltpu

_MB = 128
_NB = 40
_CAP = _MB * _NB
_SC = 512
_NC = 4
_CHUNK = 128


def _router_kernel(x_ref, rs_ref, wr_ref, pes_ref,
                   cw1_ref, cw2_ref, r1_ref, r2_ref, be_ref):
    xb = x_ref[...]
    s, d = xb.shape
    e_num = wr_ref.shape[1]
    ms = jnp.mean(jnp.square(xb), axis=-1, keepdims=True)
    rn = xb * jax.lax.rsqrt(ms + 1e-6)
    ri = rn * jax.lax.rsqrt(jnp.float32(d)) * rs_ref[...]
    logits = jnp.dot(ri, wr_ref[...], preferred_element_type=jnp.float32)
    probs = jax.nn.softmax(logits, axis=-1)
    m1 = jnp.max(logits, axis=-1, keepdims=True)
    masked = jnp.where(logits >= m1, -jnp.inf, logits)
    m2 = jnp.max(masked, axis=-1, keepdims=True)
    i1 = (logits >= m1).astype(jnp.float32)
    i2 = (masked >= m2).astype(jnp.float32)
    mask = i1 + i2
    gw = probs * mask
    rf = jnp.sum(gw, axis=-1, keepdims=True)
    rf = jnp.where(rf > 0.0, rf, 1.0)
    cw = gw / rf * pes_ref[...]
    cw1_ref[...] = jnp.sum(cw * i1, axis=-1, keepdims=True)
    cw2_ref[...] = jnp.sum(cw * i2, axis=-1, keepdims=True)

    t = i1 + i2
    counts = jnp.sum(t, axis=0, keepdims=True)
    ci = counts.astype(jnp.int32)
    pc = ((ci + _MB - 1) // _MB) * _MB
    pcf = pc.astype(jnp.float32)
    ei = jax.lax.broadcasted_iota(jnp.int32, (e_num, e_num), 0)
    ej = jax.lax.broadcasted_iota(jnp.int32, (e_num, e_num), 1)
    tri_inc = (ei <= ej).astype(jnp.float32)
    ends = jnp.dot(pcf, tri_inc, preferred_element_type=jnp.float32)
    offs = ends - pcf

    ki = jax.lax.broadcasted_iota(jnp.int32, (_CHUNK, _CHUNK), 0)
    kj = jax.lax.broadcasted_iota(jnp.int32, (_CHUNK, _CHUNK), 1)
    sl = (kj < ki).astype(jnp.float32)
    base = jnp.zeros((1, e_num), dtype=jnp.float32)
    xchunks = []
    for c in range(s // _CHUNK):
        tc = t[c * _CHUNK:(c + 1) * _CHUNK, :]
        xc = jnp.dot(sl, tc, preferred_element_type=jnp.float32) + base
        base = base + jnp.sum(tc, axis=0, keepdims=True)
        xchunks.append(xc)
    xcum = jnp.concatenate(xchunks, axis=0)
    pos = offs + xcum
    r1_ref[...] = jnp.sum(pos * i1, axis=-1, keepdims=True).astype(jnp.int32)
    r2_ref[...] = jnp.sum(pos * i2, axis=-1, keepdims=True).astype(jnp.int32)

    nbp = be_ref.shape[0]
    bpos = (jax.lax.broadcasted_iota(jnp.int32, (nbp, 1), 0) * _MB).astype(
        jnp.float32)
    be_ref[...] = jnp.sum((ends <= bpos).astype(jnp.int32), axis=-1,
                          keepdims=True)


def _gmm_kernel(be_ref, x_ref, r1_ref, r2_ref, cw1_ref, cw2_ref,
                w0_ref, w1_ref, wl_ref, out_ref):
    b = pl.program_id(0)
    s = x_ref.shape[0]
    e_num = 8

    @pl.when(be_ref[b] < e_num)
    def _expert_block():
        col = jax.lax.broadcasted_iota(jnp.int32, (s, _MB), 1) + b * _MB
        r1 = r1_ref[...]
        r2 = r2_ref[...]
        hit1 = r1 == col
        hit2 = r2 == col
        dt = hit1.astype(jnp.bfloat16) + hit2.astype(jnp.bfloat16)
        dn0 = (((0,), (0,)), ((), ()))
        sx = jax.lax.dot_general(dt, x_ref[...], dn0,
                                 preferred_element_type=jnp.float32)
        sx16 = sx.astype(jnp.bfloat16)
        w0 = w0_ref[0]
        w1 = w1_ref[0]
        wl = wl_ref[0]
        dn1 = (((1,), (1,)), ((), ()))
        h1 = jax.lax.dot_general(sx16, w0, dn1,
                                 preferred_element_type=jnp.float32)
        h2 = jax.lax.dot_general(sx16, w1, dn1,
                                 preferred_element_type=jnp.float32)
        h = (jax.nn.gelu(h1) * h2).astype(jnp.bfloat16)
        y = jax.lax.dot_general(h, wl, (((1,), (0,)), ((), ())),
                                preferred_element_type=jnp.float32)
        wmat = (jnp.where(hit1, cw1_ref[...], 0.0)
                + jnp.where(hit2, cw2_ref[...], 0.0))
        wrow = jnp.sum(wmat, axis=0, keepdims=True)
        ys16 = (y * jnp.transpose(wrow)).astype(jnp.bfloat16)
        cb = jax.lax.dot_general(dt, ys16, (((1,), (0,)), ((), ())),
                                 preferred_element_type=jnp.float32)

        @pl.when(b == 0)
        def _init():
            out_ref[...] = cb

        @pl.when(b > 0)
        def _acc():
            out_ref[...] += cb


def kernel(x, router_scale, per_expert_scale, w_router, w_gating, w_linear):
    g, s, d = x.shape
    e_num, _, f, _ = w_gating.shape
    nbp = 32

    xf = x.reshape(s, d).astype(jnp.float32)
    cw1, cw2, r1, r2, be = pl.pallas_call(
        _router_kernel,
        out_shape=(
            jax.ShapeDtypeStruct((s, 1), jnp.float32),
            jax.ShapeDtypeStruct((s, 1), jnp.float32),
            jax.ShapeDtypeStruct((s, 1), jnp.int32),
            jax.ShapeDtypeStruct((s, 1), jnp.int32),
            jax.ShapeDtypeStruct((nbp, 1), jnp.int32),
        ),
    )(xf, router_scale.reshape(1, d).astype(jnp.float32),
      w_router.astype(jnp.float32),
      per_expert_scale.reshape(1, e_num).astype(jnp.float32))

    wg16 = w_gating.astype(jnp.bfloat16)
    w0 = wg16[:, 0]
    w1 = wg16[:, 1]
    wl16 = w_linear.astype(jnp.bfloat16)
    x16 = xf.astype(jnp.bfloat16)
    bev = be.reshape(-1)

    grid_spec = pltpu.PrefetchScalarGridSpec(
        num_scalar_prefetch=1,
        grid=(_NB,),
        in_specs=[
            pl.BlockSpec((s, d), lambda b, be_s: (0, 0)),
            pl.BlockSpec((s, 1), lambda b, be_s: (0, 0)),
            pl.BlockSpec((s, 1), lambda b, be_s: (0, 0)),
            pl.BlockSpec((s, 1), lambda b, be_s: (0, 0)),
            pl.BlockSpec((s, 1), lambda b, be_s: (0, 0)),
            pl.BlockSpec((1, f, d),
                         lambda b, be_s: (jnp.minimum(be_s[b], 7), 0, 0)),
            pl.BlockSpec((1, f, d),
                         lambda b, be_s: (jnp.minimum(be_s[b], 7), 0, 0)),
            pl.BlockSpec((1, f, d),
                         lambda b, be_s: (jnp.minimum(be_s[b], 7), 0, 0)),
        ],
        out_specs=pl.BlockSpec((s, d), lambda b, be_s: (0, 0)),
    )
    out = pl.pallas_call(
        _gmm_kernel,
        grid_spec=grid_spec,
        out_shape=jax.ShapeDtypeStruct((s, d), jnp.float32),
    )(bev, x16, r1, r2, cw1, cw2, w0, w1, wl16)
    return out.reshape(g, s, d)


# speedup vs baseline: 1.3861x; 1.3861x over previous
import jax
import jax.numpy as jnp
from jax.experimental import pallas as pl
from jax.experimental.pallas import tpu as pltpu

_MB = 512
_NB = 16
_CAP = _MB * _NB
_SC = 512
_NC = 4
_CHUNK = 128


def _router_kernel(x_ref, rs_ref, wr_ref, pes_ref,
                   cw1_ref, cw2_ref, r1_ref, r2_ref, be_ref):
    xb = x_ref[...]
    s, d = xb.shape
    e_num = wr_ref.shape[1]
    ms = jnp.mean(jnp.square(xb), axis=-1, keepdims=True)
    rn = xb * jax.lax.rsqrt(ms + 1e-6)
    ri = rn * jax.lax.rsqrt(jnp.float32(d)) * rs_ref[...]
    logits = jnp.dot(ri, wr_ref[...], preferred_element_type=jnp.float32)
    probs = jax.nn.softmax(logits, axis=-1)
    m1 = jnp.max(logits, axis=-1, keepdims=True)
    masked = jnp.where(logits >= m1, -jnp.inf, logits)
    m2 = jnp.max(masked, axis=-1, keepdims=True)
    i1 = (logits >= m1).astype(jnp.float32)
    i2 = (masked >= m2).astype(jnp.float32)
    mask = i1 + i2
    gw = probs * mask
    rf = jnp.sum(gw, axis=-1, keepdims=True)
    rf = jnp.where(rf > 0.0, rf, 1.0)
    cw = gw / rf * pes_ref[...]
    cw1_ref[...] = jnp.sum(cw * i1, axis=-1, keepdims=True)
    cw2_ref[...] = jnp.sum(cw * i2, axis=-1, keepdims=True)

    t = i1 + i2
    counts = jnp.sum(t, axis=0, keepdims=True)
    ci = counts.astype(jnp.int32)
    pc = ((ci + _MB - 1) // _MB) * _MB
    pcf = pc.astype(jnp.float32)
    ei = jax.lax.broadcasted_iota(jnp.int32, (e_num, e_num), 0)
    ej = jax.lax.broadcasted_iota(jnp.int32, (e_num, e_num), 1)
    tri_inc = (ei <= ej).astype(jnp.float32)
    ends = jnp.dot(pcf, tri_inc, preferred_element_type=jnp.float32)
    offs = ends - pcf

    ki = jax.lax.broadcasted_iota(jnp.int32, (_CHUNK, _CHUNK), 0)
    kj = jax.lax.broadcasted_iota(jnp.int32, (_CHUNK, _CHUNK), 1)
    sl = (kj < ki).astype(jnp.float32)
    base = jnp.zeros((1, e_num), dtype=jnp.float32)
    xchunks = []
    for c in range(s // _CHUNK):
        tc = t[c * _CHUNK:(c + 1) * _CHUNK, :]
        xc = jnp.dot(sl, tc, preferred_element_type=jnp.float32) + base
        base = base + jnp.sum(tc, axis=0, keepdims=True)
        xchunks.append(xc)
    xcum = jnp.concatenate(xchunks, axis=0)
    pos = offs + xcum
    r1_ref[...] = jnp.sum(pos * i1, axis=-1, keepdims=True).astype(jnp.int32)
    r2_ref[...] = jnp.sum(pos * i2, axis=-1, keepdims=True).astype(jnp.int32)

    nbp = be_ref.shape[0]
    bpos = (jax.lax.broadcasted_iota(jnp.int32, (nbp, 1), 0) * _MB).astype(
        jnp.float32)
    be_ref[...] = jnp.sum((ends <= bpos).astype(jnp.int32), axis=-1,
                          keepdims=True)


def _gmm_kernel(be_ref, x_ref, r1_ref, r2_ref, cw1_ref, cw2_ref,
                w0_ref, w1_ref, wl_ref, out_ref):
    b = pl.program_id(0)
    s = x_ref.shape[0]
    e_num = 8

    @pl.when(be_ref[b] < e_num)
    def _expert_block():
        col = jax.lax.broadcasted_iota(jnp.int32, (s, _MB), 1) + b * _MB
        r1 = r1_ref[...]
        r2 = r2_ref[...]
        hit1 = r1 == col
        hit2 = r2 == col
        dt = hit1.astype(jnp.bfloat16) + hit2.astype(jnp.bfloat16)
        dn0 = (((0,), (0,)), ((), ()))
        sx = jax.lax.dot_general(dt, x_ref[...], dn0,
                                 preferred_element_type=jnp.float32)
        sx16 = sx.astype(jnp.bfloat16)
        w0 = w0_ref[0]
        w1 = w1_ref[0]
        wl = wl_ref[0]
        dn1 = (((1,), (1,)), ((), ()))
        h1 = jax.lax.dot_general(sx16, w0, dn1,
                                 preferred_element_type=jnp.float32)
        h2 = jax.lax.dot_general(sx16, w1, dn1,
                                 preferred_element_type=jnp.float32)
        h = (jax.nn.gelu(h1) * h2).astype(jnp.bfloat16)
        y = jax.lax.dot_general(h, wl, (((1,), (0,)), ((), ())),
                                preferred_element_type=jnp.float32)
        wmat = (jnp.where(hit1, cw1_ref[...], 0.0)
                + jnp.where(hit2, cw2_ref[...], 0.0))
        wrow = jnp.sum(wmat, axis=0, keepdims=True)
        ys16 = (y * jnp.transpose(wrow)).astype(jnp.bfloat16)
        cb = jax.lax.dot_general(dt, ys16, (((1,), (0,)), ((), ())),
                                 preferred_element_type=jnp.float32)

        @pl.when(b == 0)
        def _init():
            out_ref[...] = cb

        @pl.when(b > 0)
        def _acc():
            out_ref[...] += cb


def kernel(x, router_scale, per_expert_scale, w_router, w_gating, w_linear):
    g, s, d = x.shape
    e_num, _, f, _ = w_gating.shape
    nbp = 32

    xf = x.reshape(s, d).astype(jnp.float32)
    cw1, cw2, r1, r2, be = pl.pallas_call(
        _router_kernel,
        out_shape=(
            jax.ShapeDtypeStruct((s, 1), jnp.float32),
            jax.ShapeDtypeStruct((s, 1), jnp.float32),
            jax.ShapeDtypeStruct((s, 1), jnp.int32),
            jax.ShapeDtypeStruct((s, 1), jnp.int32),
            jax.ShapeDtypeStruct((nbp, 1), jnp.int32),
        ),
    )(xf, router_scale.reshape(1, d).astype(jnp.float32),
      w_router.astype(jnp.float32),
      per_expert_scale.reshape(1, e_num).astype(jnp.float32))

    wg16 = w_gating.astype(jnp.bfloat16)
    w0 = wg16[:, 0]
    w1 = wg16[:, 1]
    wl16 = w_linear.astype(jnp.bfloat16)
    x16 = xf.astype(jnp.bfloat16)
    bev = be.reshape(-1)

    grid_spec = pltpu.PrefetchScalarGridSpec(
        num_scalar_prefetch=1,
        grid=(_NB,),
        in_specs=[
            pl.BlockSpec((s, d), lambda b, be_s: (0, 0)),
            pl.BlockSpec((s, 1), lambda b, be_s: (0, 0)),
            pl.BlockSpec((s, 1), lambda b, be_s: (0, 0)),
            pl.BlockSpec((s, 1), lambda b, be_s: (0, 0)),
            pl.BlockSpec((s, 1), lambda b, be_s: (0, 0)),
            pl.BlockSpec((1, f, d),
                         lambda b, be_s: (jnp.minimum(be_s[b], 7), 0, 0)),
            pl.BlockSpec((1, f, d),
                         lambda b, be_s: (jnp.minimum(be_s[b], 7), 0, 0)),
            pl.BlockSpec((1, f, d),
                         lambda b, be_s: (jnp.minimum(be_s[b], 7), 0, 0)),
        ],
        out_specs=pl.BlockSpec((s, d), lambda b, be_s: (0, 0)),
    )
    out = pl.pallas_call(
        _gmm_kernel,
        grid_spec=grid_spec,
        out_shape=jax.ShapeDtypeStruct((s, d), jnp.float32),
    )(bev, x16, r1, r2, cw1, cw2, w0, w1, wl16)
    return out.reshape(g, s, d)
